# baseline (speedup 1.0000x reference)
import functools
import jax
import jax.numpy as jnp
from jax import lax
from jax.experimental import pallas as pl
from jax.experimental.pallas import tpu as pltpu
from jax.experimental.pallas import tpu_sc as plsc

D = 64
TEXT_VOCAB = 100000
CHAR_VOCAB = 1000
B, L = 4096, 200
NW = 32
BT = B // NW
NBC = BT // 16


def _mm_body(x_ref, w_ref, o_ref):
    y = jnp.dot(x_ref[:], w_ref[:], preferred_element_type=jnp.float32) * 8.0
    o_ref[:] = jnp.concatenate([y, jnp.zeros_like(y)], axis=1)


def _mm_bias_body(x_ref, w_ref, b_ref, o_ref):
    y = (
        jnp.dot(x_ref[:], w_ref[:], preferred_element_type=jnp.float32) * 8.0
        + b_ref[:]
    )
    o_ref[:] = jnp.concatenate([y, jnp.zeros_like(y)], axis=1)


def _fold_text_table(text_table, Wt):
    blk = 4000
    return pl.pallas_call(
        _mm_body,
        grid=(TEXT_VOCAB // blk,),
        in_specs=[
            pl.BlockSpec((blk, D), lambda i: (i, 0)),
            pl.BlockSpec((D, D), lambda i: (0, 0)),
        ],
        out_specs=pl.BlockSpec((blk, 2 * D), lambda i: (i, 0)),
        out_shape=jax.ShapeDtypeStruct((TEXT_VOCAB, 2 * D), jnp.float32),
    )(text_table, Wt)


def _fold_char_table(char_table, Wc, b2):
    return pl.pallas_call(
        _mm_bias_body,
        out_shape=jax.ShapeDtypeStruct((CHAR_VOCAB, 2 * D), jnp.float32),
    )(char_table, Wc, b2)


def _sc_body(t2_hbm, c2_hbm, idx_hbm, chars_hbm, out_hbm,
             idx_v, rowsA, rowsB, outA, outB, ct_v, cidx_v,
             semA, semB, semWA, semWB):
    wid = lax.axis_index("s") * 2 + lax.axis_index("c")
    iota = lax.iota(jnp.int32, 16)

    pltpu.sync_copy(idx_hbm.at[pl.ds(wid * L, L)], idx_v)
    pltpu.sync_copy(chars_hbm.at[pl.ds(wid * BT, BT)], cidx_v)

    pltpu.async_copy(c2_hbm.at[cidx_v], rowsA, semA).wait()

    def ct_body(bc, carry):
        sl = pl.ds(bc * 16, 16)
        slot16 = iota + bc * 16

        @plsc.parallel_loop(0, D, unroll=8, carry=jnp.zeros((16,), jnp.int32))
        def _(d, ccol):
            ct_v[d, sl] = plsc.load_gather(rowsA, [slot16, ccol])
            return ccol + 1

        return carry

    lax.fori_loop(0, NBC, ct_body, 0)

    def transpose_add_pair(bc, carry):
        sl = pl.ds(bc * 16, 16)
        slot16 = iota + bc * 16

        @plsc.parallel_loop(0, D, unroll=8, carry=jnp.zeros((16,), jnp.int32))
        def _(d, col):
            ct = ct_v[d, sl]
            outA[d, sl] = plsc.load_gather(rowsA, [slot16, col]) + ct
            outB[d, sl] = plsc.load_gather(rowsB, [slot16, col]) + ct
            return col + 1

        return carry

    out_col = pl.ds(wid * BT, BT)

    pltpu.async_copy(t2_hbm.at[idx_v.at[0]], rowsA, semA)
    pltpu.async_copy(t2_hbm.at[idx_v.at[1]], rowsB, semB)

    def body(i, carry):
        lA = 2 * i
        lB = 2 * i + 1
        pltpu.make_async_copy(t2_hbm.at[idx_v.at[0]], rowsA, semA).wait()
        pltpu.make_async_copy(t2_hbm.at[idx_v.at[0]], rowsB, semB).wait()

        @pl.when(i > 0)
        def _():
            pltpu.make_async_copy(outA, out_hbm.at[0, :, out_col], semWA).wait()
            pltpu.make_async_copy(outB, out_hbm.at[0, :, out_col], semWB).wait()

        lax.fori_loop(0, NBC, transpose_add_pair, 0)
        pltpu.async_copy(outA, out_hbm.at[lA, :, out_col], semWA)
        pltpu.async_copy(outB, out_hbm.at[lB, :, out_col], semWB)
        pltpu.async_copy(
            t2_hbm.at[idx_v.at[jnp.minimum(lA + 2, L - 1)]], rowsA, semA)
        pltpu.async_copy(
            t2_hbm.at[idx_v.at[jnp.minimum(lB + 2, L - 1)]], rowsB, semB)
        return carry

    lax.fori_loop(0, L // 2, body, 0)

    pltpu.make_async_copy(t2_hbm.at[idx_v.at[0]], rowsA, semA).wait()
    pltpu.make_async_copy(t2_hbm.at[idx_v.at[0]], rowsB, semB).wait()
    pltpu.make_async_copy(outA, out_hbm.at[0, :, out_col], semWA).wait()
    pltpu.make_async_copy(outB, out_hbm.at[0, :, out_col], semWB).wait()


def _sc_gather_transpose(T2, C2, IDX, chars):
    mesh = plsc.VectorSubcoreMesh(core_axis_name="c", subcore_axis_name="s")
    f = functools.partial(
        pl.kernel,
        mesh=mesh,
        compiler_params=pltpu.CompilerParams(
            use_tc_tiling_on_sc=True,
            needs_layout_passes=False,
            disable_bounds_checks=True,
        ),
        out_type=jax.ShapeDtypeStruct((L, D, B), jnp.float32),
        scratch_types=[
            pltpu.VMEM((L, BT), jnp.int32),
            pltpu.VMEM((BT, 128), jnp.float32),
            pltpu.VMEM((BT, 128), jnp.float32),
            pltpu.VMEM((D, BT), jnp.float32),
            pltpu.VMEM((D, BT), jnp.float32),
            pltpu.VMEM((D, BT), jnp.float32),
            pltpu.VMEM((BT,), jnp.int32),
            pltpu.SemaphoreType.DMA,
            pltpu.SemaphoreType.DMA,
            pltpu.SemaphoreType.DMA,
            pltpu.SemaphoreType.DMA,
        ],
    )(_sc_body)
    return f(T2, C2, IDX, chars)


def kernel(text_seqs, chars, text_table, char_table, W, b):
    Wt = W[:D]
    Wc = W[D:]
    T2 = _fold_text_table(text_table, Wt)
    C2 = _fold_char_table(char_table, Wc, b.reshape(1, D))
    IDX = (
        text_seqs.astype(jnp.int32)
        .reshape(NW, BT, L)
        .transpose(0, 2, 1)
        .reshape(NW * L, BT)
    )
    out_t = _sc_gather_transpose(T2, C2, IDX, chars.astype(jnp.int32))
    return out_t.transpose(2, 0, 1)


# speedup vs baseline: 1.2710x; 1.2710x over previous
import functools
import jax
import jax.numpy as jnp
from jax import lax
from jax.experimental import pallas as pl
from jax.experimental.pallas import tpu as pltpu
from jax.experimental.pallas import tpu_sc as plsc

D = 64
TEXT_VOCAB = 100000
CHAR_VOCAB = 1000
B, L = 4096, 200
NW = 32
ROWS_PER_W = B // NW
HALF = L // 2
IDX_ROWS = B * L // HALF
IDX_ROWS_PER_W = IDX_ROWS // NW


def _mm_body(x_ref, w_ref, o_ref):
    o_ref[:] = jnp.dot(x_ref[:], w_ref[:], preferred_element_type=jnp.float32) * 8.0


def _mm_bias_body(x_ref, w_ref, b_ref, o_ref):
    o_ref[:] = (
        jnp.dot(x_ref[:], w_ref[:], preferred_element_type=jnp.float32) * 8.0
        + b_ref[:]
    )


def _fold_text_table(text_table, Wt):
    blk = 4000
    return pl.pallas_call(
        _mm_body,
        grid=(TEXT_VOCAB // blk,),
        in_specs=[
            pl.BlockSpec((blk, D), lambda i: (i, 0)),
            pl.BlockSpec((D, D), lambda i: (0, 0)),
        ],
        out_specs=pl.BlockSpec((blk, D), lambda i: (i, 0)),
        out_shape=jax.ShapeDtypeStruct((TEXT_VOCAB, D), jnp.float32),
    )(text_table, Wt)


def _fold_char_table(char_table, Wc, b2):
    return pl.pallas_call(
        _mm_bias_body,
        out_shape=jax.ShapeDtypeStruct((CHAR_VOCAB, D), jnp.float32),
    )(char_table, Wc, b2)


def _sc_body(t_hbm, c_hbm, idx_hbm, chars_hbm, out_hbm,
             idx_v, cbuf_v, cidx_v, rowsA, rowsB,
             semA, semB, semWA, semWB):
    wid = lax.axis_index("s") * 2 + lax.axis_index("c")

    pltpu.sync_copy(idx_hbm.at[pl.ds(wid * IDX_ROWS_PER_W, IDX_ROWS_PER_W)], idx_v)
    pltpu.sync_copy(chars_hbm.at[pl.ds(wid * ROWS_PER_W, ROWS_PER_W)], cidx_v)
    pltpu.async_copy(c_hbm.at[cidx_v], cbuf_v, semA).wait()

    base_b = wid * ROWS_PER_W

    def fire_gather(r, rows_v, sem):
        pltpu.async_copy(t_hbm.at[idx_v.at[2 * r]], rows_v.at[pl.ds(0, HALF)], sem)
        pltpu.async_copy(
            t_hbm.at[idx_v.at[2 * r + 1]], rows_v.at[pl.ds(HALF, HALF)], sem)

    def wait_gather(rows_v, sem):
        pltpu.make_async_copy(
            t_hbm.at[idx_v.at[0]], rows_v.at[pl.ds(0, HALF)], sem).wait()
        pltpu.make_async_copy(
            t_hbm.at[idx_v.at[0]], rows_v.at[pl.ds(HALF, HALF)], sem).wait()

    def add_char(rows_v, r):
        c0 = cbuf_v[r, pl.ds(0, 16)]
        c1 = cbuf_v[r, pl.ds(16, 16)]
        c2 = cbuf_v[r, pl.ds(32, 16)]
        c3 = cbuf_v[r, pl.ds(48, 16)]

        @plsc.parallel_loop(0, L, unroll=4)
        def _(t):
            rows_v[t, pl.ds(0, 16)] = rows_v[t, pl.ds(0, 16)] + c0
            rows_v[t, pl.ds(16, 16)] = rows_v[t, pl.ds(16, 16)] + c1
            rows_v[t, pl.ds(32, 16)] = rows_v[t, pl.ds(32, 16)] + c2
            rows_v[t, pl.ds(48, 16)] = rows_v[t, pl.ds(48, 16)] + c3

    fire_gather(0, rowsA, semA)
    fire_gather(1, rowsB, semB)

    def body(i, carry):
        rA = 2 * i
        rB = 2 * i + 1
        wait_gather(rowsA, semA)

        @pl.when(i > 0)
        def _():
            pltpu.make_async_copy(rowsA, out_hbm.at[0], semWA).wait()

        add_char(rowsA, rA)
        pltpu.async_copy(rowsA, out_hbm.at[base_b + rA], semWA)
        fire_gather(jnp.minimum(rA + 2, ROWS_PER_W - 1), rowsA, semA)
        wait_gather(rowsB, semB)

        @pl.when(i > 0)
        def _():
            pltpu.make_async_copy(rowsB, out_hbm.at[0], semWB).wait()

        add_char(rowsB, rB)
        pltpu.async_copy(rowsB, out_hbm.at[base_b + rB], semWB)
        fire_gather(jnp.minimum(rB + 2, ROWS_PER_W - 1), rowsB, semB)
        return carry

    lax.fori_loop(0, ROWS_PER_W // 2, body, 0)

    wait_gather(rowsA, semA)
    wait_gather(rowsB, semB)
    pltpu.make_async_copy(rowsA, out_hbm.at[0], semWA).wait()
    pltpu.make_async_copy(rowsB, out_hbm.at[0], semWB).wait()


def _sc_gather_add(T, C, idx2, chars):
    mesh = plsc.VectorSubcoreMesh(core_axis_name="c", subcore_axis_name="s")
    f = functools.partial(
        pl.kernel,
        mesh=mesh,
        compiler_params=pltpu.CompilerParams(
            use_tc_tiling_on_sc=False,
            disable_bounds_checks=True,
        ),
        out_type=jax.ShapeDtypeStruct((B, L, D), jnp.float32),
        scratch_types=[
            pltpu.VMEM((IDX_ROWS_PER_W, HALF), jnp.int32),
            pltpu.VMEM((ROWS_PER_W, D), jnp.float32),
            pltpu.VMEM((ROWS_PER_W,), jnp.int32),
            pltpu.VMEM((L, D), jnp.float32),
            pltpu.VMEM((L, D), jnp.float32),
            pltpu.SemaphoreType.DMA,
            pltpu.SemaphoreType.DMA,
            pltpu.SemaphoreType.DMA,
            pltpu.SemaphoreType.DMA,
        ],
    )(_sc_body)
    return f(T, C, idx2, chars)


def kernel(text_seqs, chars, text_table, char_table, W, b):
    Wt = W[:D]
    Wc = W[D:]
    T = _fold_text_table(text_table, Wt)
    Cb = _fold_char_table(char_table, Wc, b.reshape(1, D))
    idx2 = text_seqs.reshape(IDX_ROWS, HALF).astype(jnp.int32)
    return _sc_gather_add(T, Cb, idx2, chars.astype(jnp.int32))
